# baseline (speedup 1.0000x reference)
import functools

import jax
import jax.numpy as jnp
import numpy as np
from jax import lax
from jax.experimental import pallas as pl
from jax.experimental.pallas import tpu as pltpu
from jax.experimental.pallas import tpu_sc as plsc

N = 1600000
B = 4096
F_S = 10
F_T = 5
W_S = 16
W_T = 16

NC = 2
NS = 16
NW = NC * NS

GROUP = 128
NGROUPS = N // GROUP
SLICE = B // NS

H = N // 2
HGROUPS = H // GROUP
GPT = 196
CHUNK_G = 2
CHUNK = GROUP * CHUNK_G
NCHUNK = GPT // CHUNK_G

R = 8000


def _repack_body(xs, xt, o1, o2):
    one = jnp.ones((R // 8, 1), jnp.float32)
    for k in range(8):
        xsk = xs[pl.ds(k * (R // 8), R // 8), :]
        o1[:, pl.ds(k * W_S, F_S + 1)] = jnp.concatenate([xsk, one], axis=1)
        xtk = xt[pl.ds(k * (R // 8), R // 8), :]
        o2[:, pl.ds(k * W_T, F_T + 1)] = jnp.concatenate([xtk, one], axis=1)


def _repack(x_s, x_t):
    grid = H // R
    return pl.pallas_call(
        _repack_body,
        grid=(grid,),
        in_specs=[pl.BlockSpec((R, F_S), lambda i: (i, 0)),
                  pl.BlockSpec((R, F_T), lambda i: (i, 0))],
        out_specs=[pl.BlockSpec((R // 8, 128), lambda i: (i, 0)),
                   pl.BlockSpec((R // 8, 128), lambda i: (i, 0))],
        out_shape=[jax.ShapeDtypeStruct((H // 8, 128), jnp.float32),
                   jax.ShapeDtypeStruct((H // 8, 128), jnp.float32)],
    )(x_s, x_t)


def _sc_segment_sums(xsw, xtw, ids_s2d, ids_t2d, z16):
    mesh = plsc.VectorSubcoreMesh(core_axis_name="c", subcore_axis_name="s")

    @functools.partial(
        pl.kernel,
        out_type=[
            jax.ShapeDtypeStruct((NC, B, W_S), jnp.float32),
            jax.ShapeDtypeStruct((NC, B, W_T), jnp.float32),
        ],
        mesh=mesh,
        compiler_params=pltpu.CompilerParams(use_tc_tiling_on_sc=False,
                                             needs_layout_passes=False),
        scratch_types=[
            pltpu.VMEM((CHUNK // 8, 128), jnp.float32),
            pltpu.VMEM((CHUNK // 8, 128), jnp.float32),
            pltpu.VMEM((CHUNK, W_S), jnp.float32),
            pltpu.VMEM((CHUNK, W_T), jnp.float32),
            pltpu.VMEM((CHUNK_G, GROUP), jnp.int32),
            pltpu.VMEM((CHUNK_G, GROUP), jnp.int32),
            pltpu.VMEM_SHARED((B, W_S), jnp.float32),
            pltpu.VMEM_SHARED((B, W_T), jnp.float32),
        ],
    )
    def seg_kernel(xsw_hbm, xtw_hbm, ids_s_hbm, ids_t_hbm, z16_hbm,
                   ps_hbm, pt_hbm,
                   ws128, wt128, ws_buf, wt_buf, ibs, ibt, acc_s, acc_t):
        core = lax.axis_index("c")
        sid = lax.axis_index("s")
        wid = sid * NC + core
        rz = sid * SLICE

        pltpu.sync_copy(z16_hbm.at[pl.ds(rz, SLICE)], acc_s.at[pl.ds(rz, SLICE)])
        pltpu.sync_copy(z16_hbm.at[pl.ds(rz, SLICE)], acc_t.at[pl.ds(rz, SLICE)])
        plsc.subcore_barrier()

        def chunk_body(c, carry):
            g0 = wid * GPT + c * CHUNK_G

            @pl.when(g0 + CHUNK_G <= HGROUPS)
            def _():
                pltpu.sync_copy(
                    xsw_hbm.at[pl.ds(g0 * (GROUP // 8), CHUNK // 8)], ws128)
                pltpu.sync_copy(
                    xtw_hbm.at[pl.ds(g0 * (GROUP // 8), CHUNK // 8)], wt128)
                pltpu.sync_copy(ids_s_hbm.at[pl.ds(g0, CHUNK_G)], ibs)
                pltpu.sync_copy(ids_t_hbm.at[pl.ds(g0, CHUNK_G)], ibt)

                def reslice(i, carry2):
                    for k in range(8):
                        ws_buf[i * 8 + k, :] = ws128[i, pl.ds(k * 16, 16)]
                        wt_buf[i * 8 + k, :] = wt128[i, pl.ds(k * 16, 16)]
                    return carry2

                lax.fori_loop(0, CHUNK // 8, reslice, 0)

                for j in range(CHUNK_G):
                    pltpu.sync_copy(ws_buf.at[pl.ds(j * GROUP, GROUP)],
                                    acc_s.at[ibs.at[j]], add=True)
                    pltpu.sync_copy(wt_buf.at[pl.ds(j * GROUP, GROUP)],
                                    acc_t.at[ibt.at[j]], add=True)

            return carry

        lax.fori_loop(0, NCHUNK, chunk_body, 0)
        plsc.subcore_barrier()

        pltpu.sync_copy(acc_s.at[pl.ds(rz, SLICE)], ps_hbm.at[core, pl.ds(rz, SLICE)])
        pltpu.sync_copy(acc_t.at[pl.ds(rz, SLICE)], pt_hbm.at[core, pl.ds(rz, SLICE)])

    return seg_kernel(xsw, xtw, ids_s2d, ids_t2d, z16)


def _mlp_body(ps, pt, u, w1a, w1b, w1c, b1, w2, b2, out):
    acc_s = jnp.sum(ps[...], axis=0)
    acc_t = jnp.sum(pt[...], axis=0)
    mean_s = acc_s[:, :F_S] / jnp.maximum(acc_s[:, F_S:F_S + 1], 1.0)
    mean_t = acc_t[:, :F_T] / jnp.maximum(acc_t[:, F_T:F_T + 1], 1.0)
    h = (jnp.dot(u[...], w1a[...], preferred_element_type=jnp.float32)
         + jnp.dot(mean_s, w1b[...], preferred_element_type=jnp.float32)
         + jnp.dot(mean_t, w1c[...], preferred_element_type=jnp.float32)
         + b1[...])
    h = jnp.where(h >= 0, h, 0.1 * h)
    out[...] = jnp.dot(h, w2[...], preferred_element_type=jnp.float32) + b2[...]


def kernel(x_s, x_t, edge_index, edge_attr, u, batch_s, batch_t, W1, b1, W2, b2):
    del edge_index, edge_attr

    def _perm(ids):
        return (ids.astype(jnp.int32)
                .reshape(H // R, 8, R // 8)
                .transpose(0, 2, 1)
                .reshape(HGROUPS, GROUP))

    z16 = jnp.zeros((B, W_S), jnp.float32)

    parts = []
    for h in range(2):
        xs_h = lax.slice_in_dim(x_s, h * H, (h + 1) * H, axis=0)
        xt_h = lax.slice_in_dim(x_t, h * H, (h + 1) * H, axis=0)
        ids_s2d = _perm(lax.slice_in_dim(batch_s, h * H, (h + 1) * H, axis=0))
        ids_t2d = _perm(lax.slice_in_dim(batch_t, h * H, (h + 1) * H, axis=0))
        xsw, xtw = _repack(xs_h, xt_h)
        parts.append(_sc_segment_sums(xsw, xtw, ids_s2d, ids_t2d, z16))
    ps = jnp.concatenate([parts[0][0], parts[1][0]], axis=0)
    pt = jnp.concatenate([parts[0][1], parts[1][1]], axis=0)

    out = pl.pallas_call(
        _mlp_body,
        out_shape=jax.ShapeDtypeStruct((B, F_S), jnp.float32),
    )(ps, pt, u,
      W1[:F_S], W1[F_S:F_S + F_S], W1[F_S + F_S:], b1.reshape(1, F_S),
      W2, b2.reshape(1, F_S))
    return out


# speedup vs baseline: 1.0688x; 1.0688x over previous
import functools

import jax
import jax.numpy as jnp
import numpy as np
from jax import lax
from jax.experimental import pallas as pl
from jax.experimental.pallas import tpu as pltpu
from jax.experimental.pallas import tpu_sc as plsc

N = 1600000
B = 4096
F_S = 10
F_T = 5
W_S = 16
W_T = 16

NC = 2
NS = 16
NW = NC * NS

GROUP = 128
GPT = 392
CHUNK_G = 4
CHUNK = GROUP * CHUNK_G
NCHUNK = GPT // CHUNK_G
NGROUPS = N // GROUP
SLICE = B // NS

R = 12800


def _repack_body(xs, xt, o1, o2):
    one = jnp.ones((R // 8, 1), jnp.float32)
    for k in range(8):
        xsk = xs[pl.ds(k * (R // 8), R // 8), :]
        o1[:, pl.ds(k * W_S, F_S + 1)] = jnp.concatenate([xsk, one], axis=1)
        xtk = xt[pl.ds(k * (R // 8), R // 8), :]
        o2[:, pl.ds(k * W_T, F_T + 1)] = jnp.concatenate([xtk, one], axis=1)


def _repack(x_s, x_t):
    grid = N // R
    return pl.pallas_call(
        _repack_body,
        grid=(grid,),
        in_specs=[pl.BlockSpec((R, F_S), lambda i: (i, 0)),
                  pl.BlockSpec((R, F_T), lambda i: (i, 0))],
        out_specs=[pl.BlockSpec((R // 8, 128), lambda i: (i, 0)),
                   pl.BlockSpec((R // 8, 128), lambda i: (i, 0))],
        out_shape=[jax.ShapeDtypeStruct((N // 8, 128), jnp.float32),
                   jax.ShapeDtypeStruct((N // 8, 128), jnp.float32)],
    )(x_s, x_t)


def _sc_segment_sums(xsw, xtw, ids_s2d, ids_t2d, z16):
    mesh = plsc.VectorSubcoreMesh(core_axis_name="c", subcore_axis_name="s")

    @functools.partial(
        pl.kernel,
        out_type=[
            jax.ShapeDtypeStruct((NC, B, W_S), jnp.float32),
            jax.ShapeDtypeStruct((NC, B, W_T), jnp.float32),
        ],
        mesh=mesh,
        compiler_params=pltpu.CompilerParams(use_tc_tiling_on_sc=False,
                                             needs_layout_passes=False),
        scratch_types=[
            pltpu.VMEM((CHUNK // 8, 128), jnp.float32),
            pltpu.VMEM((CHUNK // 8, 128), jnp.float32),
            pltpu.VMEM((CHUNK, W_S), jnp.float32),
            pltpu.VMEM((CHUNK, W_T), jnp.float32),
            pltpu.VMEM((CHUNK_G, GROUP), jnp.int32),
            pltpu.VMEM((CHUNK_G, GROUP), jnp.int32),
            pltpu.VMEM_SHARED((B, W_S), jnp.float32),
            pltpu.VMEM_SHARED((B, W_T), jnp.float32),
        ],
    )
    def seg_kernel(xsw_hbm, xtw_hbm, ids_s_hbm, ids_t_hbm, z16_hbm,
                   ps_hbm, pt_hbm,
                   ws128, wt128, ws_buf, wt_buf, ibs, ibt, acc_s, acc_t):
        core = lax.axis_index("c")
        sid = lax.axis_index("s")
        wid = sid * NC + core
        rz = sid * SLICE

        pltpu.sync_copy(z16_hbm.at[pl.ds(rz, SLICE)], acc_s.at[pl.ds(rz, SLICE)])
        pltpu.sync_copy(z16_hbm.at[pl.ds(rz, SLICE)], acc_t.at[pl.ds(rz, SLICE)])
        plsc.subcore_barrier()

        def chunk_body(c, carry):
            g0 = wid * GPT + c * CHUNK_G

            @pl.when(g0 + CHUNK_G <= NGROUPS)
            def _():
                pltpu.sync_copy(
                    xsw_hbm.at[pl.ds(g0 * (GROUP // 8), CHUNK // 8)], ws128)
                pltpu.sync_copy(
                    xtw_hbm.at[pl.ds(g0 * (GROUP // 8), CHUNK // 8)], wt128)
                pltpu.sync_copy(ids_s_hbm.at[pl.ds(g0, CHUNK_G)], ibs)
                pltpu.sync_copy(ids_t_hbm.at[pl.ds(g0, CHUNK_G)], ibt)

                def reslice(i, carry2):
                    for k in range(8):
                        ws_buf[i * 8 + k, :] = ws128[i, pl.ds(k * 16, 16)]
                        wt_buf[i * 8 + k, :] = wt128[i, pl.ds(k * 16, 16)]
                    return carry2

                lax.fori_loop(0, CHUNK // 8, reslice, 0)

                for j in range(CHUNK_G):
                    pltpu.sync_copy(ws_buf.at[pl.ds(j * GROUP, GROUP)],
                                    acc_s.at[ibs.at[j]], add=True)
                    pltpu.sync_copy(wt_buf.at[pl.ds(j * GROUP, GROUP)],
                                    acc_t.at[ibt.at[j]], add=True)

            return carry

        lax.fori_loop(0, NCHUNK, chunk_body, 0)
        plsc.subcore_barrier()

        pltpu.sync_copy(acc_s.at[pl.ds(rz, SLICE)], ps_hbm.at[core, pl.ds(rz, SLICE)])
        pltpu.sync_copy(acc_t.at[pl.ds(rz, SLICE)], pt_hbm.at[core, pl.ds(rz, SLICE)])

    return seg_kernel(xsw, xtw, ids_s2d, ids_t2d, z16)


def _mlp_body(ps, pt, u, w1a, w1b, w1c, b1, w2, b2, out):
    acc_s = ps[0] + ps[1]
    acc_t = pt[0] + pt[1]
    mean_s = acc_s[:, :F_S] / jnp.maximum(acc_s[:, F_S:F_S + 1], 1.0)
    mean_t = acc_t[:, :F_T] / jnp.maximum(acc_t[:, F_T:F_T + 1], 1.0)
    h = (jnp.dot(u[...], w1a[...], preferred_element_type=jnp.float32)
         + jnp.dot(mean_s, w1b[...], preferred_element_type=jnp.float32)
         + jnp.dot(mean_t, w1c[...], preferred_element_type=jnp.float32)
         + b1[...])
    h = jnp.where(h >= 0, h, 0.1 * h)
    out[...] = jnp.dot(h, w2[...], preferred_element_type=jnp.float32) + b2[...]


def kernel(x_s, x_t, edge_index, edge_attr, u, batch_s, batch_t, W1, b1, W2, b2):
    del edge_index, edge_attr

    def _perm(ids):
        return (ids.astype(jnp.int32)
                .reshape(N // R, 8, R // 8)
                .transpose(0, 2, 1)
                .reshape(NGROUPS, GROUP))

    ids_s2d = _perm(batch_s)
    ids_t2d = _perm(batch_t)
    z16 = jnp.zeros((B, W_S), jnp.float32)

    xsw, xtw = _repack(x_s, x_t)
    ps, pt = _sc_segment_sums(xsw, xtw, ids_s2d, ids_t2d, z16)

    out = pl.pallas_call(
        _mlp_body,
        out_shape=jax.ShapeDtypeStruct((B, F_S), jnp.float32),
    )(ps, pt, u,
      W1[:F_S], W1[F_S:F_S + F_S], W1[F_S + F_S:], b1.reshape(1, F_S),
      W2, b2.reshape(1, F_S))
    return out
